# baseline (speedup 1.0000x reference)
import functools

import jax
import jax.numpy as jnp
from jax import lax
from jax.experimental import pallas as pl
from jax.experimental.pallas import tpu as pltpu


def _conv_stats_kernel(x_ref, w2_ref, y_ref, stats_ref, p_ref, *,
                       H, W, Cin, IMG):
    HW = H * W
    lane = lax.broadcasted_iota(jnp.int32, (1, HW), 1)
    col = lane % W
    not_last = col != (W - 1)
    not_first = col != 0
    zero_edge = jnp.zeros((Cin, W), jnp.bfloat16)

    s_acc = jnp.zeros((w2_ref.shape[0], 1), jnp.float32)
    ss_acc = jnp.zeros((w2_ref.shape[0], 1), jnp.float32)
    for i in range(IMG):
        p = p_ref.at[i % 2]
        x = x_ref[i]
        xm = jnp.where(not_last, x, 0.0)
        xp = jnp.where(not_first, x, 0.0)
        for kh in range(3):
            dh = kh - 1
            for kw in range(3):
                dw = kw - 1
                t = kh * 3 + kw
                src = xm if dw == -1 else (xp if dw == 1 else x)
                delta = dh * W + dw
                shifted = pltpu.roll(src, (-delta) % HW, axis=1)
                p[t * Cin:(t + 1) * Cin, :] = shifted.astype(jnp.bfloat16)
                if dh == -1:
                    p[t * Cin:(t + 1) * Cin, 0:W] = zero_edge
                elif dh == 1:
                    p[t * Cin:(t + 1) * Cin, HW - W:HW] = zero_edge

        y = jnp.dot(w2_ref[...], p_ref[i % 2],
                    preferred_element_type=jnp.float32)
        s_acc += jnp.sum(y, axis=1, keepdims=True)
        ss_acc += jnp.sum(y * y, axis=1, keepdims=True)
        y_ref[i] = y.astype(jnp.bfloat16)
    stats_ref[...] = jnp.concatenate([s_acc, ss_acc], axis=1)


def _bn_relu_kernel(y_ref, sc_ref, sh_ref, o_ref, *, IMG):
    for i in range(IMG):
        y = y_ref[i].astype(jnp.float32)
        o_ref[i] = jnp.maximum(y * sc_ref[...] + sh_ref[...], 0.0)


def kernel(x, weight, bias, gamma, beta, *, eps=1e-5):
    N, Cin, H, W = x.shape
    Cout = weight.shape[0]
    HW = H * W
    IMG = 4 if N % 4 == 0 else (2 if N % 2 == 0 else 1)
    NB = N // IMG

    xf = x.reshape(N, Cin, HW)
    w2 = jnp.transpose(weight, (0, 2, 3, 1)).reshape(Cout, 9 * Cin)
    w2 = w2.astype(jnp.bfloat16)

    vmem_limit = 100 * 1024 * 1024

    y, stats = pl.pallas_call(
        functools.partial(_conv_stats_kernel, H=H, W=W, Cin=Cin, IMG=IMG),
        grid=(NB,),
        in_specs=[
            pl.BlockSpec((IMG, Cin, HW), lambda n: (n, 0, 0)),
            pl.BlockSpec((Cout, 9 * Cin), lambda n: (0, 0)),
        ],
        out_specs=(
            pl.BlockSpec((IMG, Cout, HW), lambda n: (n, 0, 0)),
            pl.BlockSpec((None, Cout, 2), lambda n: (n, 0, 0)),
        ),
        out_shape=(
            jax.ShapeDtypeStruct((N, Cout, HW), jnp.bfloat16),
            jax.ShapeDtypeStruct((NB, Cout, 2), jnp.float32),
        ),
        scratch_shapes=[pltpu.VMEM((2, 9 * Cin, HW), jnp.bfloat16)],
        compiler_params=pltpu.CompilerParams(
            dimension_semantics=("parallel",),
            vmem_limit_bytes=vmem_limit),
    )(xf, w2)

    count = jnp.float32(N * H * W)
    tot = jnp.sum(stats, axis=0)
    mean = tot[:, 0] / count
    var = jnp.maximum(tot[:, 1] / count - mean * mean, 0.0)
    inv = lax.rsqrt(var + eps)
    scale = (gamma * inv).reshape(Cout, 1)
    shift = (beta - mean * gamma * inv).reshape(Cout, 1)

    out = pl.pallas_call(
        functools.partial(_bn_relu_kernel, IMG=IMG),
        grid=(NB,),
        in_specs=[
            pl.BlockSpec((IMG, Cout, HW), lambda n: (n, 0, 0)),
            pl.BlockSpec((Cout, 1), lambda n: (0, 0)),
            pl.BlockSpec((Cout, 1), lambda n: (0, 0)),
        ],
        out_specs=pl.BlockSpec((IMG, Cout, HW), lambda n: (n, 0, 0)),
        out_shape=jax.ShapeDtypeStruct((N, Cout, HW), jnp.float32),
        compiler_params=pltpu.CompilerParams(
            dimension_semantics=("parallel",),
            vmem_limit_bytes=vmem_limit),
    )(y, scale, shift)

    return out.reshape(N, Cout, H, W)


# speedup vs baseline: 1.1281x; 1.1281x over previous
import functools

import jax
import jax.numpy as jnp
from jax import lax
from jax.experimental import pallas as pl
from jax.experimental.pallas import tpu as pltpu


def _conv_stats_kernel(x_ref, w3_ref, y_ref, stats_ref, p_ref, *,
                       H, W, Cin, IMG):
    HW = H * W
    Cout = w3_ref.shape[1]
    lane = lax.broadcasted_iota(jnp.int32, (1, HW), 1)
    col = lane % W
    not_last = col != (W - 1)
    not_first = col != 0
    row_lo = lane >= W
    row_hi = lane < (HW - W)

    s_acc = jnp.zeros((Cout, 1), jnp.float32)
    ss_acc = jnp.zeros((Cout, 1), jnp.float32)
    for i in range(IMG):
        p = p_ref.at[i % 2]
        x = x_ref[i]
        xm = jnp.where(not_last, x, 0.0)
        xp = jnp.where(not_first, x, 0.0)
        p[0:Cin, :] = pltpu.roll(xm, 1, axis=1).astype(jnp.bfloat16)
        p[Cin:2 * Cin, :] = x.astype(jnp.bfloat16)
        p[2 * Cin:3 * Cin, :] = pltpu.roll(xp, HW - 1, axis=1).astype(
            jnp.bfloat16)

        pv = p_ref[i % 2]
        u0 = jnp.dot(w3_ref[0], pv, preferred_element_type=jnp.float32)
        u1 = jnp.dot(w3_ref[1], pv, preferred_element_type=jnp.float32)
        u2 = jnp.dot(w3_ref[2], pv, preferred_element_type=jnp.float32)
        y = u1
        y = y + jnp.where(row_lo, pltpu.roll(u0, W, axis=1), 0.0)
        y = y + jnp.where(row_hi, pltpu.roll(u2, HW - W, axis=1), 0.0)
        s_acc += jnp.sum(y, axis=1, keepdims=True)
        ss_acc += jnp.sum(y * y, axis=1, keepdims=True)
        y_ref[i] = y.astype(jnp.bfloat16)
    stats_ref[...] = jnp.concatenate([s_acc, ss_acc], axis=1)


def _bn_relu_kernel(y_ref, sc_ref, sh_ref, o_ref, *, IMG):
    for i in range(IMG):
        y = y_ref[i].astype(jnp.float32)
        o_ref[i] = jnp.maximum(y * sc_ref[...] + sh_ref[...], 0.0)


def kernel(x, weight, bias, gamma, beta, *, eps=1e-5):
    N, Cin, H, W = x.shape
    Cout = weight.shape[0]
    HW = H * W
    IMG = 4 if N % 4 == 0 else (2 if N % 2 == 0 else 1)
    NB = N // IMG

    xf = x.reshape(N, Cin, HW)
    w3 = jnp.transpose(weight, (2, 0, 3, 1)).reshape(3, Cout, 3 * Cin)
    w3 = w3.astype(jnp.bfloat16)

    vmem_limit = 100 * 1024 * 1024

    y, stats = pl.pallas_call(
        functools.partial(_conv_stats_kernel, H=H, W=W, Cin=Cin, IMG=IMG),
        grid=(NB,),
        in_specs=[
            pl.BlockSpec((IMG, Cin, HW), lambda n: (n, 0, 0)),
            pl.BlockSpec((3, Cout, 3 * Cin), lambda n: (0, 0, 0)),
        ],
        out_specs=(
            pl.BlockSpec((IMG, Cout, HW), lambda n: (n, 0, 0)),
            pl.BlockSpec((None, Cout, 2), lambda n: (n, 0, 0)),
        ),
        out_shape=(
            jax.ShapeDtypeStruct((N, Cout, HW), jnp.bfloat16),
            jax.ShapeDtypeStruct((NB, Cout, 2), jnp.float32),
        ),
        scratch_shapes=[pltpu.VMEM((2, 3 * Cin, HW), jnp.bfloat16)],
        compiler_params=pltpu.CompilerParams(
            dimension_semantics=("parallel",),
            vmem_limit_bytes=vmem_limit),
    )(xf, w3)

    count = jnp.float32(N * H * W)
    tot = jnp.sum(stats, axis=0)
    mean = tot[:, 0] / count
    var = jnp.maximum(tot[:, 1] / count - mean * mean, 0.0)
    inv = lax.rsqrt(var + eps)
    scale = (gamma * inv).reshape(Cout, 1)
    shift = (beta - mean * gamma * inv).reshape(Cout, 1)

    out = pl.pallas_call(
        functools.partial(_bn_relu_kernel, IMG=IMG),
        grid=(NB,),
        in_specs=[
            pl.BlockSpec((IMG, Cout, HW), lambda n: (n, 0, 0)),
            pl.BlockSpec((Cout, 1), lambda n: (0, 0)),
            pl.BlockSpec((Cout, 1), lambda n: (0, 0)),
        ],
        out_specs=pl.BlockSpec((IMG, Cout, HW), lambda n: (n, 0, 0)),
        out_shape=jax.ShapeDtypeStruct((N, Cout, HW), jnp.float32),
        compiler_params=pltpu.CompilerParams(
            dimension_semantics=("parallel",),
            vmem_limit_bytes=vmem_limit),
    )(y, scale, shift)

    return out.reshape(N, Cout, H, W)
